# speedup vs baseline: 1.3479x; 1.3479x over previous
import functools

import jax
import jax.numpy as jnp
from jax.experimental import pallas as pl

_MAX_GROUP_SIZE = 4096
_CAPACITY_FACTOR = 1.25
_MIN_EXPERT_CAPACITY = 4
_TOP_K = 2


def _groups(num_tokens, max_group_size, num_experts):
    min_num_groups = max(num_tokens // max_group_size, num_experts)
    num_groups = min_num_groups
    while num_groups < num_tokens and not (
        num_tokens % num_groups == 0 and num_groups % num_experts == 0
    ):
        num_groups += 1
    return num_groups


def _routing_body(cap, x_ref, rw_ref, ei_ref, comb_ref):
    x = x_ref[0]
    t = x.shape[0]
    e = rw_ref.shape[1]
    ec = e * cap
    logits = jnp.dot(x, rw_ref[...], preferred_element_type=jnp.float32)
    probs = jax.nn.softmax(logits, axis=-1)

    eio = jax.lax.broadcasted_iota(jnp.int32, (t, e), 1)
    m0 = jnp.max(probs, axis=-1, keepdims=True)
    a0 = jnp.min(jnp.where(probs >= m0, eio, e), axis=-1, keepdims=True)
    probs1 = jnp.where(eio == a0, -1.0, probs)
    m1 = jnp.max(probs1, axis=-1, keepdims=True)
    a1 = jnp.min(jnp.where(probs1 >= m1, eio, e), axis=-1, keepdims=True)

    mask0 = (eio == a0).astype(jnp.float32)
    mask1 = (eio == a1).astype(jnp.float32)

    tr = jax.lax.broadcasted_iota(jnp.int32, (t, t), 0)
    tc = jax.lax.broadcasted_iota(jnp.int32, (t, t), 1)
    ltri = (tc <= tr).astype(jnp.float32)
    inc0 = jnp.dot(ltri, mask0, preferred_element_type=jnp.float32)
    inc1 = jnp.dot(ltri, mask1, preferred_element_type=jnp.float32)
    counts0 = jnp.sum(mask0, axis=0, keepdims=True)

    pos0 = jnp.sum(inc0 * mask0, axis=-1, keepdims=True) - 1.0
    pos1 = jnp.sum((inc1 + counts0) * mask1, axis=-1, keepdims=True) - 1.0
    pc0 = pos0.astype(jnp.int32)
    pc1 = pos1.astype(jnp.int32)
    w0 = pc0 < cap
    w1 = pc1 < cap

    ecio = jax.lax.broadcasted_iota(jnp.int32, (t, ec), 1)
    ej = ecio // cap
    cj = ecio - ej * cap
    d0 = ((ej == a0) & (cj == pc0) & w0).astype(jnp.float32)
    d1 = ((ej == a1) & (cj == pc1) & w1).astype(jnp.float32)

    disp = d0 + d1
    comb_ref[0] = m0 * d0 + m1 * d1

    ei_ref[0] = jax.lax.dot_general(
        disp, x, (((0,), (0,)), ((), ())),
        preferred_element_type=jnp.float32)


def _ffn_body(g, cap, x_ref, wi_ref, wo_ref, y_ref):
    h = x_ref.shape[-1]
    x = x_ref[...].reshape(g * cap, h)
    h1 = jnp.dot(x, wi_ref[0], preferred_element_type=jnp.float32)
    h1 = jax.nn.gelu(h1)
    y = jnp.dot(h1, wo_ref[0], preferred_element_type=jnp.float32)
    y_ref[...] = y.reshape(g, cap, h)


def _combine_body(comb_ref, y_ref, out_ref):
    out_ref[0] = jnp.dot(comb_ref[0], y_ref[0],
                         preferred_element_type=jnp.float32)


@jax.jit
def kernel(inputs, router_w, wi, wo):
    b, s, h = inputs.shape
    e = router_w.shape[1]
    f = wi.shape[2]
    num_tokens = b * s
    g = _groups(num_tokens, _MAX_GROUP_SIZE, e)
    t = num_tokens // g
    cap = max(int(round(_CAPACITY_FACTOR * t / e)), _MIN_EXPERT_CAPACITY)
    ec = e * cap

    x = inputs.reshape(g, t, h)

    ei, comb = pl.pallas_call(
        functools.partial(_routing_body, cap),
        grid=(g,),
        in_specs=[
            pl.BlockSpec((1, t, h), lambda i: (i, 0, 0)),
            pl.BlockSpec((h, e), lambda i: (0, 0)),
        ],
        out_specs=[
            pl.BlockSpec((1, ec, h), lambda i: (i, 0, 0)),
            pl.BlockSpec((1, t, ec), lambda i: (i, 0, 0)),
        ],
        out_shape=[
            jax.ShapeDtypeStruct((g, ec, h), jnp.float32),
            jax.ShapeDtypeStruct((g, t, ec), jnp.float32),
        ],
    )(x, router_w)

    y = pl.pallas_call(
        functools.partial(_ffn_body, g, cap),
        grid=(e,),
        in_specs=[
            pl.BlockSpec((g, cap, h), lambda i: (0, i, 0)),
            pl.BlockSpec((1, h, f), lambda i: (i, 0, 0)),
            pl.BlockSpec((1, f, h), lambda i: (i, 0, 0)),
        ],
        out_specs=pl.BlockSpec((g, cap, h), lambda i: (0, i, 0)),
        out_shape=jax.ShapeDtypeStruct((g, ec, h), jnp.float32),
    )(ei, wi, wo)

    out = pl.pallas_call(
        _combine_body,
        grid=(g,),
        in_specs=[
            pl.BlockSpec((1, t, ec), lambda i: (i, 0, 0)),
            pl.BlockSpec((1, ec, h), lambda i: (i, 0, 0)),
        ],
        out_specs=pl.BlockSpec((1, t, h), lambda i: (i, 0, 0)),
        out_shape=jax.ShapeDtypeStruct((g, t, h), jnp.float32),
    )(comb, y)

    return out.reshape(b, s, h)
